# baseline (speedup 1.0000x reference)
import functools

import jax
import jax.numpy as jnp
from jax import lax
from jax.experimental import pallas as pl
from jax.experimental.pallas import tpu as pltpu
from jax.experimental.pallas import tpu_sc as plsc

N0, N1, N2 = 10000, 2500, 625
E = 160000
H = 128
A = 4
AH = A * H

NC, NS = 2, 16
NW = NC * NS

N1P = 2560
N2P = 640

EP = 163840
MGN = 32768

Y0 = 0.28209479177387814
C1 = 0.48860251190291987

f32 = jnp.float32


@functools.lru_cache(maxsize=None)
def _gather_fn(V, D, M):
    assert M % (NW * 1024) == 0 and D == 128
    per = M // NW
    nr = per // 128
    mesh = plsc.VectorSubcoreMesh(core_axis_name="c", subcore_axis_name="s")

    def body(tab, idx2, out, iball, b0, b1, b2, b3, gsem, wsem):
        wid = lax.axis_index("s") * NC + lax.axis_index("c")
        base = pl.multiple_of(wid * per, 1024)
        rbase = pl.multiple_of(wid * nr, 8)
        pltpu.sync_copy(idx2.at[pl.ds(rbase, nr)], iball)
        bufs = [b0, b1, b2, b3]
        gd = [None] * 4
        wr = [None] * 4

        def emit_write(jj):
            bb = jj % 4
            gd[bb].wait()
            wr[bb] = pltpu.async_copy(
                bufs[bb], out.at[pl.ds(base + jj * 128, 128)], wsem)

        for j in range(nr):
            b = j % 4
            if wr[b] is not None:
                wr[b].wait()
                wr[b] = None
            gd[b] = pltpu.async_copy(tab.at[iball.at[j]], bufs[b], gsem)
            if j >= 2:
                emit_write(j - 2)
        for jj in range(max(0, nr - 2), nr):
            emit_write(jj)
        for b in range(4):
            if wr[b] is not None:
                wr[b].wait()

    return pl.kernel(
        body,
        out_type=jax.ShapeDtypeStruct((M, D), f32),
        mesh=mesh,
        scratch_types=[
            pltpu.VMEM((nr, 128), jnp.int32),
            pltpu.VMEM((128, D), f32),
            pltpu.VMEM((128, D), f32),
            pltpu.VMEM((128, D), f32),
            pltpu.VMEM((128, D), f32),
            pltpu.SemaphoreType.DMA,
            pltpu.SemaphoreType.DMA,
        ],
    )


def _gather(table, idx):
    V, D = table.shape
    (M,) = idx.shape
    return _gather_fn(V, D, M)(table, idx.reshape(M // 128, 128))


@functools.lru_cache(maxsize=None)
def _gather_multi_fn(Vs, M, pairs, ni):
    nt = len(Vs)
    per = M // NW
    nr = per // 128
    mesh = plsc.VectorSubcoreMesh(core_axis_name="c", subcore_axis_name="s")

    def body(*refs):
        tabs = refs[:nt]
        idxs = refs[nt:nt + ni]
        outs = refs[nt + ni:nt + ni + len(pairs)]
        sc = refs[nt + ni + len(pairs):]
        iballs = sc[:ni]
        bufs = sc[ni:ni + 6]
        gsem, wsem = sc[ni + 6], sc[ni + 7]
        wid = lax.axis_index("s") * NC + lax.axis_index("c")
        base = pl.multiple_of(wid * per, 1024)
        rbase = pl.multiple_of(wid * nr, 8)
        for ii in range(ni):
            pltpu.sync_copy(idxs[ii].at[pl.ds(rbase, nr)], iballs[ii])
        wr = [None] * 6
        pend = []
        step = 0

        def flush_one():
            d, bb, o, off = pend.pop(0)
            d.wait()
            wr[bb] = pltpu.async_copy(bufs[bb], o.at[pl.ds(off, 128)], wsem)

        for ti, ii in pairs:
            for j in range(nr):
                b = step % 6
                if wr[b] is not None:
                    wr[b].wait()
                    wr[b] = None
                gd = pltpu.async_copy(tabs[ti].at[iballs[ii].at[j]],
                                      bufs[b], gsem)
                pend.append((gd, b, outs[pairs.index((ti, ii))],
                             base + j * 128))
                if len(pend) > 3:
                    flush_one()
                step += 1
        while pend:
            flush_one()
        for b in range(6):
            if wr[b] is not None:
                wr[b].wait()

    return pl.kernel(
        body,
        out_type=[jax.ShapeDtypeStruct((M, 128), f32)] * len(pairs),
        mesh=mesh,
        scratch_types=(
            [pltpu.VMEM((nr, 128), jnp.int32)] * ni
            + [pltpu.VMEM((128, 128), f32)] * 6
            + [pltpu.SemaphoreType.DMA, pltpu.SemaphoreType.DMA]
        ),
    )


def _gather_sd(tabs, idxS, idxD):
    (M,) = idxS.shape
    nt = len(tabs)
    pairs = tuple((t, i) for i in range(2) for t in range(nt))
    outs = _gather_multi_fn(tuple(t.shape[0] for t in tabs), M, pairs, 2)(
        *tabs, idxS.reshape(M // 128, 128), idxD.reshape(M // 128, 128))
    return list(outs[:nt]), list(outs[nt:])


@functools.lru_cache(maxsize=None)
def _scatter_fn(M, NPs):
    D = 128
    assert M % (NW * 1024) == 0 and NPs % 128 == 0
    per = M // NW
    nr = per // 128
    pr = NPs // NS
    mesh = plsc.VectorSubcoreMesh(core_axis_name="c", subcore_axis_name="s")

    def body(vals, idx2, zeros, out, acc, iball, v0, v1, lsem, ssem):
        cid = lax.axis_index("c")
        sid = lax.axis_index("s")
        row0 = pl.multiple_of(sid * pr, 8)
        pltpu.sync_copy(zeros.at[pl.ds(row0, pr)], acc.at[pl.ds(row0, pr)])
        base = pl.multiple_of(cid * (M // 2) + sid * per, 1024)
        rbase = pl.multiple_of(cid * (M // 256) + sid * nr, 8)
        pltpu.sync_copy(idx2.at[pl.ds(rbase, nr)], iball)
        plsc.subcore_barrier()
        vbufs = [v0, v1]
        ld = [None, None]
        sc = [None, None]

        def emit_scatter(jj):
            bb = jj % 2
            ld[bb].wait()
            sc[bb] = pltpu.async_copy(vbufs[bb], acc.at[iball.at[jj]],
                                      ssem, add=True)

        for j in range(nr):
            b = j % 2
            if sc[b] is not None:
                sc[b].wait()
                sc[b] = None
            ld[b] = pltpu.async_copy(
                vals.at[pl.ds(base + j * 128, 128)], vbufs[b], lsem)
            if j >= 1:
                emit_scatter(j - 1)
        emit_scatter(nr - 1)
        for b in range(2):
            if sc[b] is not None:
                sc[b].wait()
        plsc.subcore_barrier()
        pltpu.sync_copy(acc.at[pl.ds(row0, pr)],
                        out.at[cid, pl.ds(row0, pr)])

    return pl.kernel(
        body,
        out_type=jax.ShapeDtypeStruct((2, NPs, D), f32),
        mesh=mesh,
        scratch_types=[
            pltpu.VMEM_SHARED((NPs, D), f32),
            pltpu.VMEM((nr, 128), jnp.int32),
            pltpu.VMEM((128, D), f32),
            pltpu.VMEM((128, D), f32),
            pltpu.SemaphoreType.DMA,
            pltpu.SemaphoreType.DMA,
        ],
    )


def _scatter_add(vals, idx, NP, raw=False):
    M, D = vals.shape
    assert D == 128
    NPs = (NP + 127) // 128 * 128
    out = _scatter_fn(M, NPs)(vals, idx.reshape(M // 128, 128),
                              jnp.zeros((NPs, D), f32))
    if raw:
        return out
    return out[0][:NP], out[1][:NP]


def _swish(v):
    return v * jax.nn.sigmoid(v)


@functools.lru_cache(maxsize=None)
def _geom_fn(M, B):
    nb = M // B

    def body(ps_ref, pd_ref, out_ref, pad_ref):
        d = ps_ref[:, 0:3] - pd_ref[:, 0:3]
        l2 = jnp.sum(d * d, axis=1, keepdims=True)
        l = jnp.sqrt(l2)
        n = d * (C1 / (l + 1e-8))
        one = jnp.ones((B, 1), f32)
        g = jnp.concatenate(
            [Y0 * one, n, l, one, jnp.zeros((B, 10), f32)], axis=1)
        if M > E:
            i = pl.program_id(0)
            rows = i * B + lax.broadcasted_iota(jnp.int32, (B, 1), 0)
            g = jnp.where(rows < E, g, 0.0)
        out_ref[...] = g
        pad_ref[...] = jnp.concatenate([g, jnp.zeros((B, 112), f32)], axis=1)

    return pl.pallas_call(
        body,
        grid=(nb,),
        in_specs=[pl.BlockSpec((B, 128), lambda i: (i, 0))] * 2,
        out_specs=[pl.BlockSpec((B, 16), lambda i: (i, 0)),
                   pl.BlockSpec((B, 128), lambda i: (i, 0))],
        out_shape=[jax.ShapeDtypeStruct((M, 16), f32),
                   jax.ShapeDtypeStruct((M, 128), f32)],
    )


@functools.lru_cache(maxsize=None)
def _edge_fn(nparts, M, B):
    nb = M // B

    def body(*refs):
        geom_ref = refs[0]
        part_refs = refs[1:1 + nparts]
        w1_refs = refs[1 + nparts:1 + 2 * nparts]
        w1len_ref = refs[1 + 2 * nparts]
        w2_ref = refs[2 + 2 * nparts]
        out_ref = refs[3 + 2 * nparts]

        g = geom_ref[...]
        l = g[:, 4:5]
        P = l * w1len_ref[0:1, :]
        for p_ref, w_ref in zip(part_refs, w1_refs):
            P = P + jnp.dot(p_ref[...], w_ref[...],
                            preferred_element_type=f32)
        m1 = jnp.zeros((B, H), f32)
        for a in range(A):
            m1 = m1 + g[:, a:a + 1] * P[:, a * H:(a + 1) * H]
        m1 = _swish(m1)
        P2 = jnp.dot(m1, w2_ref[...], preferred_element_type=f32)
        m2 = jnp.zeros((B, H), f32)
        for a in range(A):
            m2 = m2 + g[:, a:a + 1] * P2[:, a * H:(a + 1) * H]
        m2 = _swish(m2)
        if M > E:
            i = pl.program_id(0)
            rows = i * B + lax.broadcasted_iota(jnp.int32, (B, 1), 0)
            m2 = jnp.where(rows < E, m2, 0.0)
        out_ref[...] = m2

    in_specs = (
        [pl.BlockSpec((B, 16), lambda i: (i, 0))]
        + [pl.BlockSpec((B, H), lambda i: (i, 0))] * nparts
        + [pl.BlockSpec((H, AH), lambda i: (0, 0))] * nparts
        + [pl.BlockSpec((8, AH), lambda i: (0, 0))]
        + [pl.BlockSpec((H, AH), lambda i: (0, 0))]
    )
    return pl.pallas_call(
        body,
        grid=(nb,),
        in_specs=in_specs,
        out_specs=pl.BlockSpec((B, H), lambda i: (i, 0)),
        out_shape=jax.ShapeDtypeStruct((M, H), f32),
    )


@functools.lru_cache(maxsize=None)
def _update_fn(nparts, NP, B, n_real, residual, stats, final, pair=False):
    nb = NP // B
    nw_extra = 2 if final else 0
    nw = nparts - 1 if pair else nparts

    def body(*refs):
        part_refs = refs[:nparts]
        attr_ref = refs[nparts]
        w_refs = refs[nparts + 1:nparts + nw + 1]
        idx = nparts + nw + 1
        if final:
            wamb1_ref, wamb2_ref = refs[idx], refs[idx + 1]
            idx += 2
        out_ref = refs[idx]
        g = attr_ref[...]

        def tp(v, w_ref):
            Pv = jnp.dot(v, w_ref[...], preferred_element_type=f32)
            r = jnp.zeros((B, H), f32)
            for a in range(A):
                r = r + g[:, a:a + 1] * Pv[:, a * H:(a + 1) * H]
            return r

        P = jnp.zeros((B, AH), f32)
        if pair:
            vals = [r[...] for r in part_refs[:nw - 1]]
            vals.append(part_refs[nw - 1][...] + part_refs[nw][...])
        else:
            vals = [r[...] for r in part_refs]
        for v, w_ref in zip(vals, w_refs):
            P = P + jnp.dot(v, w_ref[...], preferred_element_type=f32)
        y = jnp.zeros((B, H), f32)
        for a in range(A):
            y = y + g[:, a:a + 1] * P[:, a * H:(a + 1) * H]
        if residual:
            y = y + part_refs[0][...]
        if final:
            y = _swish(tp(y, wamb1_ref))
            y = tp(y, wamb2_ref)
        if n_real < NP:
            i = pl.program_id(0)
            rows = i * B + lax.broadcasted_iota(jnp.int32, (B, 1), 0)
            y = jnp.where(rows < n_real, y, 0.0)
        out_ref[...] = y
        if stats:
            refs[idx + 1][...] = jnp.sum(y, axis=0)[None, None, :]
            refs[idx + 2][...] = jnp.sum(y * y, axis=0)[None, None, :]

    in_specs = (
        [pl.BlockSpec((B, H), lambda i: (i, 0))] * nparts
        + [pl.BlockSpec((B, 16), lambda i: (i, 0))]
        + [pl.BlockSpec((H, AH), lambda i: (0, 0))] * (nw + nw_extra)
    )
    out_shape = [jax.ShapeDtypeStruct((NP, H), f32)]
    out_specs = [pl.BlockSpec((B, H), lambda i: (i, 0))]
    if stats:
        out_shape += [jax.ShapeDtypeStruct((nb, 1, H), f32)] * 2
        out_specs += [pl.BlockSpec((1, 1, H), lambda i: (i, 0, 0))] * 2
    return pl.pallas_call(
        body,
        grid=(nb,),
        in_specs=in_specs,
        out_specs=out_specs,
        out_shape=out_shape,
    )


@functools.lru_cache(maxsize=None)
def _norm_fn(NP, B, n_real):
    nb = NP // B

    def body(y_ref, s_ref, q_ref, out_ref):
        s = jnp.sum(s_ref[...], axis=(0, 1))
        q = jnp.sum(q_ref[...], axis=(0, 1))
        mu = s / n_real
        var = jnp.maximum(q / n_real - mu * mu, 0.0)
        sd = jnp.sqrt(var) + 1e-5
        out = (y_ref[...] - mu[None, :]) / sd[None, :]
        if n_real < NP:
            i = pl.program_id(0)
            rows = i * B + lax.broadcasted_iota(jnp.int32, (B, 1), 0)
            out = jnp.where(rows < n_real, out, 0.0)
        out_ref[...] = out

    return pl.pallas_call(
        body,
        grid=(nb,),
        in_specs=[
            pl.BlockSpec((B, H), lambda i: (i, 0)),
            pl.BlockSpec((nb, 1, H), lambda i: (0, 0, 0)),
            pl.BlockSpec((nb, 1, H), lambda i: (0, 0, 0)),
        ],
        out_specs=pl.BlockSpec((B, H), lambda i: (i, 0)),
        out_shape=jax.ShapeDtypeStruct((NP, H), f32),
    )


@functools.lru_cache(maxsize=None)
def _attr_fin_fn(NP, B):
    nb = NP // B

    def body(s0_ref, s1_ref, out_ref):
        s = s0_ref[...] + s1_ref[...]
        cnt = jnp.maximum(s[:, 5:6], 1.0)
        out_ref[...] = jnp.concatenate(
            [s[:, 0:4] / cnt, jnp.zeros((B, 12), f32)], axis=1)

    return pl.pallas_call(
        body,
        grid=(nb,),
        in_specs=[pl.BlockSpec((B, 128), lambda i: (i, 0))] * 2,
        out_specs=pl.BlockSpec((B, 16), lambda i: (i, 0)),
        out_shape=jax.ShapeDtypeStruct((NP, 16), f32),
    )


@functools.lru_cache(maxsize=None)
def _pool_fin_fn(NP, B, n_real):
    nb = NP // B

    def body(xs0_ref, xs1_ref, ps0_ref, ps1_ref, xc_ref, pc_ref):
        ps = ps0_ref[...] + ps1_ref[...]
        cnt = jnp.maximum(ps[:, 3:4], 1.0)
        xc_ref[...] = (xs0_ref[...] + xs1_ref[...]) / cnt
        pc = ps / cnt
        i = pl.program_id(0)
        rows = i * B + lax.broadcasted_iota(jnp.int32, (B, 1), 0)
        one = jnp.where(rows < n_real, 1.0, 0.0)
        pc_ref[...] = jnp.concatenate([pc[:, 0:3], one, pc[:, 4:]], axis=1)

    return pl.pallas_call(
        body,
        grid=(nb,),
        in_specs=[pl.BlockSpec((B, H), lambda i: (i, 0))] * 2
        + [pl.BlockSpec((B, 128), lambda i: (i, 0))] * 2,
        out_specs=[
            pl.BlockSpec((B, H), lambda i: (i, 0)),
            pl.BlockSpec((B, 128), lambda i: (i, 0)),
        ],
        out_shape=[
            jax.ShapeDtypeStruct((NP, H), f32),
            jax.ShapeDtypeStruct((NP, 128), f32),
        ],
    )


@functools.lru_cache(maxsize=None)
def _oh_gather_fn(NT, VT, B):
    nb = NT // B
    CH = 512 if VT > 512 else VT

    def body(cl_ref, tab_ref, out_ref):
        cl = cl_ref[0, 0, :][:, None]
        acc = jnp.zeros((B, H), f32)
        for c0 in range(0, VT, CH):
            w = min(CH, VT - c0)
            ids = c0 + lax.broadcasted_iota(jnp.int32, (B, w), 1)
            oh = (cl == ids).astype(f32)
            acc = acc + jnp.dot(oh, tab_ref[c0:c0 + w, :],
                                preferred_element_type=f32)
        out_ref[...] = acc

    return pl.pallas_call(
        body,
        grid=(nb,),
        in_specs=[
            pl.BlockSpec((1, 1, B), lambda i: (i, 0, 0)),
            pl.BlockSpec((VT, H), lambda i: (0, 0)),
        ],
        out_specs=pl.BlockSpec((B, H), lambda i: (i, 0)),
        out_shape=jax.ShapeDtypeStruct((NT, H), f32),
    )


def _oh_gather(cl, tab, NT, B):
    return _oh_gather_fn(NT, tab.shape[0], B)(cl.reshape(NT // B, 1, B), tab)


@functools.lru_cache(maxsize=None)
def _oh_reagg_fn(NPc, Bc, NSrc, Bk, pair):
    nc, nk = NPc // Bc, NSrc // Bk

    def body(*refs):
        if pair:
            cl_ref, s0_ref, s1_ref, out_ref = refs
            sv = s0_ref[...] + s1_ref[...]
        else:
            cl_ref, s0_ref, out_ref = refs
            sv = s0_ref[...]
        i = pl.program_id(0)
        k = pl.program_id(1)
        cl = cl_ref[0, 0, :][None, :]
        ids = i * Bc + lax.broadcasted_iota(jnp.int32, (Bc, Bk), 0)
        ohT = (ids == cl).astype(f32)
        contrib = jnp.dot(ohT, sv, preferred_element_type=f32)

        @pl.when(k == 0)
        def _():
            out_ref[...] = contrib

        @pl.when(k > 0)
        def _():
            out_ref[...] = out_ref[...] + contrib

    in_specs = [pl.BlockSpec((1, 1, Bk), lambda i, k: (k, 0, 0))]
    in_specs += [pl.BlockSpec((Bk, H), lambda i, k: (k, 0))] * (2 if pair else 1)
    return pl.pallas_call(
        body,
        grid=(nc, nk),
        in_specs=in_specs,
        out_specs=pl.BlockSpec((Bc, H), lambda i, k: (i, 0)),
        out_shape=jax.ShapeDtypeStruct((NPc, H), f32),
    )


@functools.lru_cache(maxsize=None)
def _oh_pool_fn(NPc, Bc, NSrc, Bk, n_real_c, n_src_real):
    nc, nk = NPc // Bc, NSrc // Bk

    def body(cl_ref, x_ref, p_ref, xc_ref, pc_ref, cnt_ref):
        i = pl.program_id(0)
        k = pl.program_id(1)
        cl = cl_ref[0, 0, :][None, :]
        ids = i * Bc + lax.broadcasted_iota(jnp.int32, (Bc, Bk), 0)
        cols = k * Bk + lax.broadcasted_iota(jnp.int32, (Bc, Bk), 1)
        ohT = ((ids == cl) & (cols < n_src_real)).astype(f32)
        xs = jnp.dot(ohT, x_ref[...], preferred_element_type=f32)
        ps = jnp.dot(ohT, p_ref[...], preferred_element_type=f32)
        cn = jnp.sum(ohT, axis=1, keepdims=True)

        @pl.when(k == 0)
        def _():
            xc_ref[...] = xs
            pc_ref[...] = ps
            cnt_ref[...] = cn

        @pl.when(k > 0)
        def _():
            xc_ref[...] = xc_ref[...] + xs
            pc_ref[...] = pc_ref[...] + ps
            cnt_ref[...] = cnt_ref[...] + cn

        @pl.when(k == nk - 1)
        def _():
            cnt = jnp.maximum(cnt_ref[...], 1.0)
            xc_ref[...] = xc_ref[...] / cnt
            pc = pc_ref[...] / cnt
            rows = i * Bc + lax.broadcasted_iota(jnp.int32, (Bc, 1), 0)
            one = jnp.where(rows < n_real_c, 1.0, 0.0)
            pc_ref[...] = jnp.concatenate([pc[:, 0:3], one, pc[:, 4:]],
                                          axis=1)

    return pl.pallas_call(
        body,
        grid=(nc, nk),
        in_specs=[
            pl.BlockSpec((1, 1, Bk), lambda i, k: (k, 0, 0)),
            pl.BlockSpec((Bk, H), lambda i, k: (k, 0)),
            pl.BlockSpec((Bk, H), lambda i, k: (k, 0)),
        ],
        out_specs=[
            pl.BlockSpec((Bc, H), lambda i, k: (i, 0)),
            pl.BlockSpec((Bc, H), lambda i, k: (i, 0)),
        ],
        out_shape=[
            jax.ShapeDtypeStruct((NPc, H), f32),
            jax.ShapeDtypeStruct((NPc, H), f32),
        ],
        scratch_shapes=[pltpu.VMEM((Bc, 1), f32)],
    )


@functools.lru_cache(maxsize=None)
def _attr_fin1_fn(NP, B):
    nb = NP // B

    def body(s_ref, out_ref):
        sv = s_ref[...]
        cnt = jnp.maximum(sv[:, 5:6], 1.0)
        out_ref[...] = jnp.concatenate(
            [sv[:, 0:4] / cnt, jnp.zeros((B, 12), f32)], axis=1)

    return pl.pallas_call(
        body,
        grid=(nb,),
        in_specs=[pl.BlockSpec((B, 128), lambda i: (i, 0))],
        out_specs=pl.BlockSpec((B, 16), lambda i: (i, 0)),
        out_shape=jax.ShapeDtypeStruct((NP, 16), f32),
    )


def _wprep(W, part_rows):
    d_in = W.shape[0] // A
    Wc = W.reshape(d_in, A, H).reshape(d_in, AH)
    parts = [Wc[r:r + H] for r in part_rows]
    wlen = None
    if d_in % H == 1:
        wlen = jnp.pad(Wc[d_in - 1:d_in], ((0, 7), (0, 0)))
    return parts, wlen


def _edge_block(m):
    return 1000 if m % 1000 == 0 else 640


def _layer(src_gath, dst_gath, node_parts, geom, nattr, seg,
           NP, n_real, Wm1, Wm2, Wu, residual, norm, BN,
           final_ws=None):
    parts = src_gath + dst_gath
    np_ = len(parts)
    w1p, w1len = _wprep(Wm1, [i * H for i in range(np_)])
    w2p, _ = _wprep(Wm2, [0])
    m = _edge_fn(np_, EP, 1024)(geom, *parts, *w1p, w1len, w2p[0])
    agg = seg(m)
    pair = isinstance(agg, tuple)
    nparts = node_parts + (list(agg) if pair else [agg])
    wup, _ = _wprep(Wu, [i * H for i in range(len(node_parts) + 1)])
    stats = norm
    args = nparts + [nattr] + wup
    if final_ws is not None:
        a1, _ = _wprep(final_ws[0], [0])
        a2, _ = _wprep(final_ws[1], [0])
        args += [a1[0], a2[0]]
    res = _update_fn(len(nparts), NP, BN, n_real, residual,
                     stats, final_ws is not None, pair=pair)(*args)
    if stats:
        y, st, q = res
        return _norm_fn(NP, BN, n_real)(y, st, q)
    return res[0]


def _o3(ps, pd, seg, NP, BN):
    geom, gpad = _geom_fn(EP, 1024)(ps, pd)
    asum = seg(gpad)
    if isinstance(asum, tuple):
        nattr = _attr_fin_fn(NP, BN)(*asum)
    else:
        nattr = _attr_fin1_fn(NP, BN)(asum)
    return geom, nattr


def kernel(x, pos, edge_index, batch, cluster0, cluster1, params):
    del batch
    p = params
    eip = jnp.pad(edge_index, ((0, 0), (0, EP - E)))
    src0 = eip[0]
    dst0 = eip[1]

    pos0 = jnp.concatenate(
        [pos, jnp.ones((N0, 1), f32), jnp.zeros((N0, 124), f32)], axis=1)

    NPS0 = 10240
    cl0w = jnp.pad(cluster0, (0, NPS0 - N0))
    cl1p = jnp.pad(cluster1, (0, N1P - N1))

    def seg0(m):
        return _scatter_add(m, dst0, N0)

    cl0w3 = cl0w.reshape(NPS0 // 1024, 1, 1024)
    cl1p3 = cl1p.reshape(N1P // 512, 1, 512)

    def seg1(m):
        r = _scatter_add(m, dst0, NPS0, raw=True)
        return _oh_reagg_fn(N1P, 512, NPS0, 1024, True)(cl0w3, r[0], r[1])

    def seg2(m):
        return _oh_reagg_fn(N2P, 640, N1P, 512, False)(cl1p3, seg1(m))

    ps0, pd0 = _gather_sd([pos0], src0, dst0)
    geom0, nattr0 = _o3(ps0[0], pd0[0], seg0, N0, 1000)
    we, _ = _wprep(p['W_emb'], [0])
    x0 = _update_fn(1, N0, 1000, N0, False, False, False)(x, nattr0, we[0])[0]

    g0s, g0d = _gather_sd([x0], src0, dst0)
    x0 = _layer(g0s, g0d, [x0],
                geom0, nattr0, seg0, N0, N0,
                p['Wm1_0'], p['Wm2_0'], p['Wu_0'], True, True, 1000)
    copy0 = x0

    x1, pos1 = _oh_pool_fn(N1P, 512, N0, 1000, N1, N0)(cluster0.reshape(10, 1, 1000), x0, pos0)
    P1 = _oh_gather(cluster0, pos1, N0, 1000)
    p1s, p1d = _gather_sd([P1], src0, dst0)
    geom1, nattr1 = _o3(p1s[0], p1d[0], seg1, N1P, 512)

    T1 = _oh_gather(cluster0, x1, N0, 1000)
    g1s, g1d = _gather_sd([T1], src0, dst0)
    x1 = _layer(g1s, g1d, [x1],
                geom1, nattr1, seg1, N1P, N1,
                p['Wm1_1'], p['Wm2_1'], p['Wu_1'], True, True, 512)
    copy1 = x1

    x2, pos2 = _oh_pool_fn(N2P, 640, N1P, 512, N2, N1)(cl1p.reshape(5, 1, 512), x1, pos1)
    P2 = _oh_gather(cluster0, _oh_gather(cl1p, pos2, N1P, 512), N0, 1000)
    p2s, p2d = _gather_sd([P2], src0, dst0)
    geom2, nattr2 = _o3(p2s[0], p2d[0], seg2, N2P, 640)

    for i in (2, 3, 4):
        T = _oh_gather(cluster0, _oh_gather(cl1p, x2, N1P, 512), N0, 1000)
        gts, gtd = _gather_sd([T], src0, dst0)
        x2 = _layer(gts, gtd, [x2],
                    geom2, nattr2, seg2, N2P, N2,
                    p['Wm1_%d' % i], p['Wm2_%d' % i], p['Wu_%d' % i],
                    True, True, 640)

    U5 = _oh_gather(cl1p, x2, N1P, 512)
    TA = _oh_gather(cluster0, U5, N0, 1000)
    TB = _oh_gather(cluster0, copy1, N0, 1000)
    g5s, g5d = _gather_sd([TA, TB], src0, dst0)
    x1 = _layer(g5s, g5d,
                [U5, copy1],
                geom1, nattr1, seg1, N1P, N1,
                p['Wm1_5'], p['Wm2_5'], p['Wu_5'], False, True, 512)

    T6 = _oh_gather(cluster0, x1, N0, 1000)
    g6s, g6d = _gather_sd([T6, copy0], src0, dst0)
    x0 = _layer(g6s, g6d,
                [T6, copy0],
                geom0, nattr0, seg0, N0, N0,
                p['Wm1_6'], p['Wm2_6'], p['Wu_6'], False, True, 1000)

    g7s, g7d = _gather_sd([x0], src0, dst0)
    x0 = _layer(g7s, g7d, [x0],
                geom0, nattr0, seg0, N0, N0,
                p['Wm1_7'], p['Wm2_7'], p['Wu_7'], True, False, 1000,
                final_ws=(p['W_amb1'], p['W_amb2']))
    return x0


# speedup vs baseline: 1.0536x; 1.0536x over previous
import functools

import jax
import jax.numpy as jnp
from jax import lax
from jax.experimental import pallas as pl
from jax.experimental.pallas import tpu as pltpu
from jax.experimental.pallas import tpu_sc as plsc

N0, N1, N2 = 10000, 2500, 625
E = 160000
H = 128
A = 4
AH = A * H

NC, NS = 2, 16
NW = NC * NS

N1P = 2560
N2P = 640

EP = 163840

Y0 = 0.28209479177387814
C1 = 0.48860251190291987

f32 = jnp.float32


@functools.lru_cache(maxsize=None)
def _gather_fn(V, D, M):
    assert M % (NW * 1024) == 0 and D == 128
    per = M // NW
    nr = per // 128
    mesh = plsc.VectorSubcoreMesh(core_axis_name="c", subcore_axis_name="s")

    def body(tab, idx2, out, iball, b0, b1, b2, b3, gsem, wsem):
        wid = lax.axis_index("s") * NC + lax.axis_index("c")
        base = pl.multiple_of(wid * per, 1024)
        rbase = pl.multiple_of(wid * nr, 8)
        pltpu.sync_copy(idx2.at[pl.ds(rbase, nr)], iball)
        bufs = [b0, b1, b2, b3]
        gd = [None] * 4
        wr = [None] * 4

        def emit_write(jj):
            bb = jj % 4
            gd[bb].wait()
            wr[bb] = pltpu.async_copy(
                bufs[bb], out.at[pl.ds(base + jj * 128, 128)], wsem)

        for j in range(nr):
            b = j % 4
            if wr[b] is not None:
                wr[b].wait()
                wr[b] = None
            gd[b] = pltpu.async_copy(tab.at[iball.at[j]], bufs[b], gsem)
            if j >= 2:
                emit_write(j - 2)
        for jj in range(max(0, nr - 2), nr):
            emit_write(jj)
        for b in range(4):
            if wr[b] is not None:
                wr[b].wait()

    return pl.kernel(
        body,
        out_type=jax.ShapeDtypeStruct((M, D), f32),
        mesh=mesh,
        scratch_types=[
            pltpu.VMEM((nr, 128), jnp.int32),
            pltpu.VMEM((128, D), f32),
            pltpu.VMEM((128, D), f32),
            pltpu.VMEM((128, D), f32),
            pltpu.VMEM((128, D), f32),
            pltpu.SemaphoreType.DMA,
            pltpu.SemaphoreType.DMA,
        ],
    )


def _gather(table, idx):
    V, D = table.shape
    (M,) = idx.shape
    return _gather_fn(V, D, M)(table, idx.reshape(M // 128, 128))


@functools.lru_cache(maxsize=None)
def _scatter_fn(M, NPs):
    D = 128
    assert M % (NW * 1024) == 0 and NPs % 128 == 0
    per = M // NW
    nr = per // 128
    pr = NPs // NS
    mesh = plsc.VectorSubcoreMesh(core_axis_name="c", subcore_axis_name="s")

    def body(vals, idx2, zeros, out, acc, iball, v0, v1, lsem, ssem):
        cid = lax.axis_index("c")
        sid = lax.axis_index("s")
        row0 = pl.multiple_of(sid * pr, 8)
        pltpu.sync_copy(zeros.at[pl.ds(row0, pr)], acc.at[pl.ds(row0, pr)])
        base = pl.multiple_of(cid * (M // 2) + sid * per, 1024)
        rbase = pl.multiple_of(cid * (M // 256) + sid * nr, 8)
        pltpu.sync_copy(idx2.at[pl.ds(rbase, nr)], iball)
        plsc.subcore_barrier()
        vbufs = [v0, v1]
        ld = [None, None]
        sc = [None, None]

        def emit_scatter(jj):
            bb = jj % 2
            ld[bb].wait()
            sc[bb] = pltpu.async_copy(vbufs[bb], acc.at[iball.at[jj]],
                                      ssem, add=True)

        for j in range(nr):
            b = j % 2
            if sc[b] is not None:
                sc[b].wait()
                sc[b] = None
            ld[b] = pltpu.async_copy(
                vals.at[pl.ds(base + j * 128, 128)], vbufs[b], lsem)
            if j >= 1:
                emit_scatter(j - 1)
        emit_scatter(nr - 1)
        for b in range(2):
            if sc[b] is not None:
                sc[b].wait()
        plsc.subcore_barrier()
        pltpu.sync_copy(acc.at[pl.ds(row0, pr)],
                        out.at[cid, pl.ds(row0, pr)])

    return pl.kernel(
        body,
        out_type=jax.ShapeDtypeStruct((2, NPs, D), f32),
        mesh=mesh,
        scratch_types=[
            pltpu.VMEM_SHARED((NPs, D), f32),
            pltpu.VMEM((nr, 128), jnp.int32),
            pltpu.VMEM((128, D), f32),
            pltpu.VMEM((128, D), f32),
            pltpu.SemaphoreType.DMA,
            pltpu.SemaphoreType.DMA,
        ],
    )


def _scatter_add(vals, idx, NP, raw=False):
    M, D = vals.shape
    assert D == 128
    NPs = (NP + 127) // 128 * 128
    out = _scatter_fn(M, NPs)(vals, idx.reshape(M // 128, 128),
                              jnp.zeros((NPs, D), f32))
    if raw:
        return out
    return out[0][:NP], out[1][:NP]


def _swish(v):
    return v * jax.nn.sigmoid(v)


@functools.lru_cache(maxsize=None)
def _geom_fn(M, B):
    nb = M // B

    def body(ps_ref, pd_ref, out_ref, pad_ref):
        d = ps_ref[:, 0:3] - pd_ref[:, 0:3]
        l2 = jnp.sum(d * d, axis=1, keepdims=True)
        l = jnp.sqrt(l2)
        n = d * (C1 / (l + 1e-8))
        one = jnp.ones((B, 1), f32)
        g = jnp.concatenate(
            [Y0 * one, n, l, one, jnp.zeros((B, 10), f32)], axis=1)
        if M > E:
            i = pl.program_id(0)
            rows = i * B + lax.broadcasted_iota(jnp.int32, (B, 1), 0)
            g = jnp.where(rows < E, g, 0.0)
        out_ref[...] = g
        pad_ref[...] = jnp.concatenate([g, jnp.zeros((B, 112), f32)], axis=1)

    return pl.pallas_call(
        body,
        grid=(nb,),
        in_specs=[pl.BlockSpec((B, 128), lambda i: (i, 0))] * 2,
        out_specs=[pl.BlockSpec((B, 16), lambda i: (i, 0)),
                   pl.BlockSpec((B, 128), lambda i: (i, 0))],
        out_shape=[jax.ShapeDtypeStruct((M, 16), f32),
                   jax.ShapeDtypeStruct((M, 128), f32)],
    )


@functools.lru_cache(maxsize=None)
def _edge_fn(nparts, M, B):
    nb = M // B

    def body(*refs):
        geom_ref = refs[0]
        part_refs = refs[1:1 + nparts]
        w1_refs = refs[1 + nparts:1 + 2 * nparts]
        w1len_ref = refs[1 + 2 * nparts]
        w2_ref = refs[2 + 2 * nparts]
        out_ref = refs[3 + 2 * nparts]

        g = geom_ref[...]
        l = g[:, 4:5]
        P = l * w1len_ref[0:1, :]
        for p_ref, w_ref in zip(part_refs, w1_refs):
            P = P + jnp.dot(p_ref[...], w_ref[...],
                            preferred_element_type=f32)
        m1 = jnp.zeros((B, H), f32)
        for a in range(A):
            m1 = m1 + g[:, a:a + 1] * P[:, a * H:(a + 1) * H]
        m1 = _swish(m1)
        P2 = jnp.dot(m1, w2_ref[...], preferred_element_type=f32)
        m2 = jnp.zeros((B, H), f32)
        for a in range(A):
            m2 = m2 + g[:, a:a + 1] * P2[:, a * H:(a + 1) * H]
        m2 = _swish(m2)
        if M > E:
            i = pl.program_id(0)
            rows = i * B + lax.broadcasted_iota(jnp.int32, (B, 1), 0)
            m2 = jnp.where(rows < E, m2, 0.0)
        out_ref[...] = m2

    in_specs = (
        [pl.BlockSpec((B, 16), lambda i: (i, 0))]
        + [pl.BlockSpec((B, H), lambda i: (i, 0))] * nparts
        + [pl.BlockSpec((H, AH), lambda i: (0, 0))] * nparts
        + [pl.BlockSpec((8, AH), lambda i: (0, 0))]
        + [pl.BlockSpec((H, AH), lambda i: (0, 0))]
    )
    return pl.pallas_call(
        body,
        grid=(nb,),
        in_specs=in_specs,
        out_specs=pl.BlockSpec((B, H), lambda i: (i, 0)),
        out_shape=jax.ShapeDtypeStruct((M, H), f32),
    )


@functools.lru_cache(maxsize=None)
def _update_fn(nparts, NP, B, n_real, residual, stats, final, pair=False):
    nb = NP // B
    nw_extra = 2 if final else 0
    nw = nparts - 1 if pair else nparts

    def body(*refs):
        part_refs = refs[:nparts]
        attr_ref = refs[nparts]
        w_refs = refs[nparts + 1:nparts + nw + 1]
        idx = nparts + nw + 1
        if final:
            wamb1_ref, wamb2_ref = refs[idx], refs[idx + 1]
            idx += 2
        out_ref = refs[idx]
        g = attr_ref[...]

        def tp(v, w_ref):
            Pv = jnp.dot(v, w_ref[...], preferred_element_type=f32)
            r = jnp.zeros((B, H), f32)
            for a in range(A):
                r = r + g[:, a:a + 1] * Pv[:, a * H:(a + 1) * H]
            return r

        P = jnp.zeros((B, AH), f32)
        if pair:
            vals = [r[...] for r in part_refs[:nw - 1]]
            vals.append(part_refs[nw - 1][...] + part_refs[nw][...])
        else:
            vals = [r[...] for r in part_refs]
        for v, w_ref in zip(vals, w_refs):
            P = P + jnp.dot(v, w_ref[...], preferred_element_type=f32)
        y = jnp.zeros((B, H), f32)
        for a in range(A):
            y = y + g[:, a:a + 1] * P[:, a * H:(a + 1) * H]
        if residual:
            y = y + part_refs[0][...]
        if final:
            y = _swish(tp(y, wamb1_ref))
            y = tp(y, wamb2_ref)
        if n_real < NP:
            i = pl.program_id(0)
            rows = i * B + lax.broadcasted_iota(jnp.int32, (B, 1), 0)
            y = jnp.where(rows < n_real, y, 0.0)
        out_ref[...] = y
        if stats:
            refs[idx + 1][...] = jnp.sum(y, axis=0)[None, None, :]
            refs[idx + 2][...] = jnp.sum(y * y, axis=0)[None, None, :]

    in_specs = (
        [pl.BlockSpec((B, H), lambda i: (i, 0))] * nparts
        + [pl.BlockSpec((B, 16), lambda i: (i, 0))]
        + [pl.BlockSpec((H, AH), lambda i: (0, 0))] * (nw + nw_extra)
    )
    out_shape = [jax.ShapeDtypeStruct((NP, H), f32)]
    out_specs = [pl.BlockSpec((B, H), lambda i: (i, 0))]
    if stats:
        out_shape += [jax.ShapeDtypeStruct((nb, 1, H), f32)] * 2
        out_specs += [pl.BlockSpec((1, 1, H), lambda i: (i, 0, 0))] * 2
    return pl.pallas_call(
        body,
        grid=(nb,),
        in_specs=in_specs,
        out_specs=out_specs,
        out_shape=out_shape,
    )


@functools.lru_cache(maxsize=None)
def _norm_fn(NP, B, n_real):
    nb = NP // B

    def body(y_ref, s_ref, q_ref, out_ref):
        s = jnp.sum(s_ref[...], axis=(0, 1))
        q = jnp.sum(q_ref[...], axis=(0, 1))
        mu = s / n_real
        var = jnp.maximum(q / n_real - mu * mu, 0.0)
        sd = jnp.sqrt(var) + 1e-5
        out = (y_ref[...] - mu[None, :]) / sd[None, :]
        if n_real < NP:
            i = pl.program_id(0)
            rows = i * B + lax.broadcasted_iota(jnp.int32, (B, 1), 0)
            out = jnp.where(rows < n_real, out, 0.0)
        out_ref[...] = out

    return pl.pallas_call(
        body,
        grid=(nb,),
        in_specs=[
            pl.BlockSpec((B, H), lambda i: (i, 0)),
            pl.BlockSpec((nb, 1, H), lambda i: (0, 0, 0)),
            pl.BlockSpec((nb, 1, H), lambda i: (0, 0, 0)),
        ],
        out_specs=pl.BlockSpec((B, H), lambda i: (i, 0)),
        out_shape=jax.ShapeDtypeStruct((NP, H), f32),
    )


@functools.lru_cache(maxsize=None)
def _attr_fin_fn(NP, B):
    nb = NP // B

    def body(s0_ref, s1_ref, out_ref):
        s = s0_ref[...] + s1_ref[...]
        cnt = jnp.maximum(s[:, 5:6], 1.0)
        out_ref[...] = jnp.concatenate(
            [s[:, 0:4] / cnt, jnp.zeros((B, 12), f32)], axis=1)

    return pl.pallas_call(
        body,
        grid=(nb,),
        in_specs=[pl.BlockSpec((B, 128), lambda i: (i, 0))] * 2,
        out_specs=pl.BlockSpec((B, 16), lambda i: (i, 0)),
        out_shape=jax.ShapeDtypeStruct((NP, 16), f32),
    )


@functools.lru_cache(maxsize=None)
def _pool_fin_fn(NP, B, n_real):
    nb = NP // B

    def body(xs0_ref, xs1_ref, ps0_ref, ps1_ref, xc_ref, pc_ref):
        ps = ps0_ref[...] + ps1_ref[...]
        cnt = jnp.maximum(ps[:, 3:4], 1.0)
        xc_ref[...] = (xs0_ref[...] + xs1_ref[...]) / cnt
        pc = ps / cnt
        i = pl.program_id(0)
        rows = i * B + lax.broadcasted_iota(jnp.int32, (B, 1), 0)
        one = jnp.where(rows < n_real, 1.0, 0.0)
        pc_ref[...] = jnp.concatenate([pc[:, 0:3], one, pc[:, 4:]], axis=1)

    return pl.pallas_call(
        body,
        grid=(nb,),
        in_specs=[pl.BlockSpec((B, H), lambda i: (i, 0))] * 2
        + [pl.BlockSpec((B, 128), lambda i: (i, 0))] * 2,
        out_specs=[
            pl.BlockSpec((B, H), lambda i: (i, 0)),
            pl.BlockSpec((B, 128), lambda i: (i, 0)),
        ],
        out_shape=[
            jax.ShapeDtypeStruct((NP, H), f32),
            jax.ShapeDtypeStruct((NP, 128), f32),
        ],
    )


@functools.lru_cache(maxsize=None)
def _oh_gather_fn(NT, VT, B):
    nb = NT // B
    CH = 512 if VT > 512 else VT

    def body(cl_ref, tab_ref, out_ref):
        cl = cl_ref[0, 0, :][:, None]
        acc = jnp.zeros((B, H), f32)
        for c0 in range(0, VT, CH):
            w = min(CH, VT - c0)
            ids = c0 + lax.broadcasted_iota(jnp.int32, (B, w), 1)
            oh = (cl == ids).astype(f32)
            acc = acc + jnp.dot(oh, tab_ref[c0:c0 + w, :],
                                preferred_element_type=f32)
        out_ref[...] = acc

    return pl.pallas_call(
        body,
        grid=(nb,),
        in_specs=[
            pl.BlockSpec((1, 1, B), lambda i: (i, 0, 0)),
            pl.BlockSpec((VT, H), lambda i: (0, 0)),
        ],
        out_specs=pl.BlockSpec((B, H), lambda i: (i, 0)),
        out_shape=jax.ShapeDtypeStruct((NT, H), f32),
    )


def _oh_gather(cl, tab, NT, B):
    return _oh_gather_fn(NT, tab.shape[0], B)(cl.reshape(NT // B, 1, B), tab)


@functools.lru_cache(maxsize=None)
def _oh_reagg_fn(NPc, Bc, NSrc, Bk, pair):
    nc, nk = NPc // Bc, NSrc // Bk

    def body(*refs):
        if pair:
            cl_ref, s0_ref, s1_ref, out_ref = refs
            sv = s0_ref[...] + s1_ref[...]
        else:
            cl_ref, s0_ref, out_ref = refs
            sv = s0_ref[...]
        i = pl.program_id(0)
        k = pl.program_id(1)
        cl = cl_ref[0, 0, :][None, :]
        ids = i * Bc + lax.broadcasted_iota(jnp.int32, (Bc, Bk), 0)
        ohT = (ids == cl).astype(f32)
        contrib = jnp.dot(ohT, sv, preferred_element_type=f32)

        @pl.when(k == 0)
        def _():
            out_ref[...] = contrib

        @pl.when(k > 0)
        def _():
            out_ref[...] = out_ref[...] + contrib

    in_specs = [pl.BlockSpec((1, 1, Bk), lambda i, k: (k, 0, 0))]
    in_specs += [pl.BlockSpec((Bk, H), lambda i, k: (k, 0))] * (2 if pair else 1)
    return pl.pallas_call(
        body,
        grid=(nc, nk),
        in_specs=in_specs,
        out_specs=pl.BlockSpec((Bc, H), lambda i, k: (i, 0)),
        out_shape=jax.ShapeDtypeStruct((NPc, H), f32),
    )


@functools.lru_cache(maxsize=None)
def _oh_pool_fn(NPc, Bc, NSrc, Bk, n_real_c, n_src_real):
    nc, nk = NPc // Bc, NSrc // Bk

    def body(cl_ref, x_ref, p_ref, xc_ref, pc_ref, cnt_ref):
        i = pl.program_id(0)
        k = pl.program_id(1)
        cl = cl_ref[0, 0, :][None, :]
        ids = i * Bc + lax.broadcasted_iota(jnp.int32, (Bc, Bk), 0)
        cols = k * Bk + lax.broadcasted_iota(jnp.int32, (Bc, Bk), 1)
        ohT = ((ids == cl) & (cols < n_src_real)).astype(f32)
        xs = jnp.dot(ohT, x_ref[...], preferred_element_type=f32)
        ps = jnp.dot(ohT, p_ref[...], preferred_element_type=f32)
        cn = jnp.sum(ohT, axis=1, keepdims=True)

        @pl.when(k == 0)
        def _():
            xc_ref[...] = xs
            pc_ref[...] = ps
            cnt_ref[...] = cn

        @pl.when(k > 0)
        def _():
            xc_ref[...] = xc_ref[...] + xs
            pc_ref[...] = pc_ref[...] + ps
            cnt_ref[...] = cnt_ref[...] + cn

        @pl.when(k == nk - 1)
        def _():
            cnt = jnp.maximum(cnt_ref[...], 1.0)
            xc_ref[...] = xc_ref[...] / cnt
            pc = pc_ref[...] / cnt
            rows = i * Bc + lax.broadcasted_iota(jnp.int32, (Bc, 1), 0)
            one = jnp.where(rows < n_real_c, 1.0, 0.0)
            pc_ref[...] = jnp.concatenate([pc[:, 0:3], one, pc[:, 4:]],
                                          axis=1)

    return pl.pallas_call(
        body,
        grid=(nc, nk),
        in_specs=[
            pl.BlockSpec((1, 1, Bk), lambda i, k: (k, 0, 0)),
            pl.BlockSpec((Bk, H), lambda i, k: (k, 0)),
            pl.BlockSpec((Bk, H), lambda i, k: (k, 0)),
        ],
        out_specs=[
            pl.BlockSpec((Bc, H), lambda i, k: (i, 0)),
            pl.BlockSpec((Bc, H), lambda i, k: (i, 0)),
        ],
        out_shape=[
            jax.ShapeDtypeStruct((NPc, H), f32),
            jax.ShapeDtypeStruct((NPc, H), f32),
        ],
        scratch_shapes=[pltpu.VMEM((Bc, 1), f32)],
    )


@functools.lru_cache(maxsize=None)
def _attr_fin1_fn(NP, B):
    nb = NP // B

    def body(s_ref, out_ref):
        sv = s_ref[...]
        cnt = jnp.maximum(sv[:, 5:6], 1.0)
        out_ref[...] = jnp.concatenate(
            [sv[:, 0:4] / cnt, jnp.zeros((B, 12), f32)], axis=1)

    return pl.pallas_call(
        body,
        grid=(nb,),
        in_specs=[pl.BlockSpec((B, 128), lambda i: (i, 0))],
        out_specs=pl.BlockSpec((B, 16), lambda i: (i, 0)),
        out_shape=jax.ShapeDtypeStruct((NP, 16), f32),
    )


def _wprep(W, part_rows):
    d_in = W.shape[0] // A
    Wc = W.reshape(d_in, A, H).reshape(d_in, AH)
    parts = [Wc[r:r + H] for r in part_rows]
    wlen = None
    if d_in % H == 1:
        wlen = jnp.pad(Wc[d_in - 1:d_in], ((0, 7), (0, 0)))
    return parts, wlen


def _layer(src_gath, dst_gath, node_parts, geom, nattr, seg,
           NP, n_real, Wm1, Wm2, Wu, residual, norm, BN,
           final_ws=None):
    parts = src_gath + dst_gath
    np_ = len(parts)
    w1p, w1len = _wprep(Wm1, [i * H for i in range(np_)])
    w2p, _ = _wprep(Wm2, [0])
    m = _edge_fn(np_, EP, 1024)(geom, *parts, *w1p, w1len, w2p[0])
    agg = seg(m)
    pair = isinstance(agg, tuple)
    nparts = node_parts + (list(agg) if pair else [agg])
    wup, _ = _wprep(Wu, [i * H for i in range(len(node_parts) + 1)])
    stats = norm
    args = nparts + [nattr] + wup
    if final_ws is not None:
        a1, _ = _wprep(final_ws[0], [0])
        a2, _ = _wprep(final_ws[1], [0])
        args += [a1[0], a2[0]]
    res = _update_fn(len(nparts), NP, BN, n_real, residual,
                     stats, final_ws is not None, pair=pair)(*args)
    if stats:
        y, st, q = res
        return _norm_fn(NP, BN, n_real)(y, st, q)
    return res[0]


def _o3(ps, pd, seg, NP, BN):
    geom, gpad = _geom_fn(EP, 1024)(ps, pd)
    asum = seg(gpad)
    if isinstance(asum, tuple):
        nattr = _attr_fin_fn(NP, BN)(*asum)
    else:
        nattr = _attr_fin1_fn(NP, BN)(asum)
    return geom, nattr


def kernel(x, pos, edge_index, batch, cluster0, cluster1, params):
    del batch
    p = params
    eip = jnp.pad(edge_index, ((0, 0), (0, EP - E)))
    src0 = eip[0]
    dst0 = eip[1]

    pos0 = jnp.concatenate(
        [pos, jnp.ones((N0, 1), f32), jnp.zeros((N0, 124), f32)], axis=1)

    NPS0 = 10240
    cl0w = jnp.pad(cluster0, (0, NPS0 - N0))
    cl1p = jnp.pad(cluster1, (0, N1P - N1))

    def seg0(m):
        return _scatter_add(m, dst0, N0)

    cl0w3 = cl0w.reshape(NPS0 // 1024, 1, 1024)
    cl1p3 = cl1p.reshape(N1P // 512, 1, 512)

    def seg1(m):
        r = _scatter_add(m, dst0, NPS0, raw=True)
        return _oh_reagg_fn(N1P, 512, NPS0, 1024, True)(cl0w3, r[0], r[1])

    def seg2(m):
        return _oh_reagg_fn(N2P, 640, N1P, 512, False)(cl1p3, seg1(m))

    geom0, nattr0 = _o3(_gather(pos0, src0), _gather(pos0, dst0),
                        seg0, N0, 1000)
    we, _ = _wprep(p['W_emb'], [0])
    x0 = _update_fn(1, N0, 1000, N0, False, False, False)(x, nattr0, we[0])[0]

    x0 = _layer([_gather(x0, src0)], [_gather(x0, dst0)], [x0],
                geom0, nattr0, seg0, N0, N0,
                p['Wm1_0'], p['Wm2_0'], p['Wu_0'], True, True, 1000)
    copy0 = x0

    x1, pos1 = _oh_pool_fn(N1P, 512, N0, 1000, N1, N0)(cluster0.reshape(10, 1, 1000), x0, pos0)
    P1 = _oh_gather(cluster0, pos1, N0, 1000)
    geom1, nattr1 = _o3(_gather(P1, src0), _gather(P1, dst0),
                        seg1, N1P, 512)

    T1 = _oh_gather(cluster0, x1, N0, 1000)
    x1 = _layer([_gather(T1, src0)], [_gather(T1, dst0)], [x1],
                geom1, nattr1, seg1, N1P, N1,
                p['Wm1_1'], p['Wm2_1'], p['Wu_1'], True, True, 512)
    copy1 = x1

    x2, pos2 = _oh_pool_fn(N2P, 640, N1P, 512, N2, N1)(cl1p.reshape(5, 1, 512), x1, pos1)
    P2 = _oh_gather(cluster0, _oh_gather(cl1p, pos2, N1P, 512), N0, 1000)
    geom2, nattr2 = _o3(_gather(P2, src0), _gather(P2, dst0),
                        seg2, N2P, 640)

    for i in (2, 3, 4):
        T = _oh_gather(cluster0, _oh_gather(cl1p, x2, N1P, 512), N0, 1000)
        x2 = _layer([_gather(T, src0)], [_gather(T, dst0)], [x2],
                    geom2, nattr2, seg2, N2P, N2,
                    p['Wm1_%d' % i], p['Wm2_%d' % i], p['Wu_%d' % i],
                    True, True, 640)

    U5 = _oh_gather(cl1p, x2, N1P, 512)
    TA = _oh_gather(cluster0, U5, N0, 1000)
    TB = _oh_gather(cluster0, copy1, N0, 1000)
    x1 = _layer([_gather(TA, src0), _gather(TB, src0)],
                [_gather(TA, dst0), _gather(TB, dst0)],
                [U5, copy1],
                geom1, nattr1, seg1, N1P, N1,
                p['Wm1_5'], p['Wm2_5'], p['Wu_5'], False, True, 512)

    T6 = _oh_gather(cluster0, x1, N0, 1000)
    x0 = _layer([_gather(T6, src0), _gather(copy0, src0)],
                [_gather(T6, dst0), _gather(copy0, dst0)],
                [T6, copy0],
                geom0, nattr0, seg0, N0, N0,
                p['Wm1_6'], p['Wm2_6'], p['Wu_6'], False, True, 1000)

    x0 = _layer([_gather(x0, src0)], [_gather(x0, dst0)], [x0],
                geom0, nattr0, seg0, N0, N0,
                p['Wm1_7'], p['Wm2_7'], p['Wu_7'], True, False, 1000,
                final_ws=(p['W_amb1'], p['W_amb2']))
    return x0
